# baseline (speedup 1.0000x reference)
import functools

import jax
import jax.numpy as jnp
from jax import lax
from jax.experimental import pallas as pl
from jax.experimental.pallas import tpu as pltpu
from jax.experimental.pallas import tpu_sc as plsc

N = 10000
E = 320000
NC, NS, L = 2, 16, 16
NW = NC * NS
NPW = 320
NPAD = NW * NPW
G = 128
FIRE = 3 * G
NGRP = 8
CAP = FIRE + NGRP * L
CHUNK = 6400
NEG = -3.0e38

BN = 1000


def _norm_pool_body(x_ref, wp_ref, bp_ref, h_ref, hp_ref):
    x = x_ref[...]
    ss = jnp.sum(x * x, axis=1, keepdims=True)
    nrm = jnp.maximum(jnp.sqrt(ss), 1e-12)
    h = x / nrm
    h_ref[...] = h
    acc = jnp.dot(h, wp_ref[...], preferred_element_type=jnp.float32)
    hp_ref[...] = jnp.maximum(acc + bp_ref[...], 0.0)


def _norm_pool(x, wp, bp):
    return pl.pallas_call(
        _norm_pool_body,
        out_shape=[jax.ShapeDtypeStruct((N, 128), jnp.float32),
                   jax.ShapeDtypeStruct((N, 128), jnp.float32)],
        grid=(N // BN,),
        in_specs=[
            pl.BlockSpec((BN, 128), lambda i: (i, 0)),
            pl.BlockSpec((128, 128), lambda i: (0, 0)),
            pl.BlockSpec((1, 128), lambda i: (0, 0)),
        ],
        out_specs=[pl.BlockSpec((BN, 128), lambda i: (i, 0)),
                   pl.BlockSpec((BN, 128), lambda i: (i, 0))],
    )(x, wp, bp)


def _combine_pool_body(h_ref, a_ref, ws_ref, wn_ref, b_ref, wp_ref, bp_ref,
                       o_ref, hp_ref):
    a = a_ref[...]
    a = jnp.where(a < -1e30, 0.0, a)
    acc = jnp.dot(h_ref[...], ws_ref[...], preferred_element_type=jnp.float32)
    acc += jnp.dot(a, wn_ref[...], preferred_element_type=jnp.float32)
    hn = jnp.maximum(acc + b_ref[...], 0.0)
    o_ref[...] = hn
    acc2 = jnp.dot(hn, wp_ref[...], preferred_element_type=jnp.float32)
    hp_ref[...] = jnp.maximum(acc2 + bp_ref[...], 0.0)


def _combine_pool(h, agg, ws, wn, b, wp, bp):
    dpi = h.shape[1]
    dact = agg.shape[1]
    dpo = ws.shape[1]
    return pl.pallas_call(
        _combine_pool_body,
        out_shape=[jax.ShapeDtypeStruct((N, dpo), jnp.float32),
                   jax.ShapeDtypeStruct((N, 128), jnp.float32)],
        grid=(N // BN,),
        in_specs=[
            pl.BlockSpec((BN, dpi), lambda i: (i, 0)),
            pl.BlockSpec((BN, dact), lambda i: (i, 0)),
            pl.BlockSpec((dpi, dpo), lambda i: (0, 0)),
            pl.BlockSpec((dact, dpo), lambda i: (0, 0)),
            pl.BlockSpec((1, dpo), lambda i: (0, 0)),
            pl.BlockSpec((dpo, 128), lambda i: (0, 0)),
            pl.BlockSpec((1, 128), lambda i: (0, 0)),
        ],
        out_specs=[pl.BlockSpec((BN, dpo), lambda i: (i, 0)),
                   pl.BlockSpec((BN, 128), lambda i: (i, 0))],
    )(h, agg, ws, wn, b, wp, bp)


def _combine_body(h_ref, a_ref, ws_ref, wn_ref, b_ref, o_ref, *, relu):
    a = a_ref[...]
    a = jnp.where(a < -1e30, 0.0, a)
    acc = jnp.dot(h_ref[...], ws_ref[...], preferred_element_type=jnp.float32)
    acc += jnp.dot(a, wn_ref[...], preferred_element_type=jnp.float32)
    acc += b_ref[...]
    if relu:
        acc = jnp.maximum(acc, 0.0)
    o_ref[...] = acc


def _combine(h, agg, ws, wn, b, relu):
    dpi = h.shape[1]
    dact = agg.shape[1]
    dpo = ws.shape[1]
    return pl.pallas_call(
        functools.partial(_combine_body, relu=relu),
        out_shape=jax.ShapeDtypeStruct((N, dpo), jnp.float32),
        grid=(N // BN,),
        in_specs=[
            pl.BlockSpec((BN, dpi), lambda i: (i, 0)),
            pl.BlockSpec((BN, dact), lambda i: (i, 0)),
            pl.BlockSpec((dpi, dpo), lambda i: (0, 0)),
            pl.BlockSpec((dact, dpo), lambda i: (0, 0)),
            pl.BlockSpec((1, dpo), lambda i: (0, 0)),
        ],
        out_specs=pl.BlockSpec((BN, dpo), lambda i: (i, 0)),
    )(h, agg, ws, wn, b)


_MESH = plsc.VectorSubcoreMesh(
    core_axis_name="c", subcore_axis_name="s", num_cores=NC, num_subcores=NS)
_SC_PARAMS = pltpu.CompilerParams(needs_layout_passes=False)


def _wid():
    return lax.axis_index("s") * NC + lax.axis_index("c")


NCHUNK = E // CHUNK


def _partition_body(adj_hbm, part_hbm, cnt_hbm,
                    ad0_v, ad1_v, sel_p, cnt_v, sem0, sem1):
    wid = _wid()
    lo = wid * NPW
    hi = lo + NPW
    ad_v = (ad0_v, ad1_v)
    sems = (sem0, sem1)

    for i in range(CAP // L):
        sel_p[pl.ds(i * L, L)] = jnp.zeros((L,), jnp.int32)

    def grp_body(ad, g, carry):
        nselv, nsel, nfired = carry
        base = g * NGRP * L
        s4 = [ad[0, pl.ds(base + k * L, L)] for k in range(NGRP)]
        d4 = [ad[1, pl.ds(base + k * L, L)] for k in range(NGRP)]
        m4 = [(d >= lo) & (d < hi) for d in d4]
        cum4 = [plsc.cumsum(m.astype(jnp.int32)) for m in m4]
        pc4 = [plsc.all_reduce_population_count(m) for m in m4]
        pk4 = [s | ((d - lo) << 16) for s, d in zip(s4, d4)]
        off = nselv - 1
        for k in range(NGRP):
            plsc.store_scatter(sel_p, [off + cum4[k]], pk4[k], mask=m4[k])
            off = off + pc4[k]
        tot = pc4[0]
        for k in range(1, NGRP):
            tot = tot + pc4[k]
        nselv = nselv + tot
        nsel = nsel + tot[0]

        full = nsel >= FIRE

        @pl.when(full)
        def _fire():
            pltpu.sync_copy(sel_p.at[pl.ds(0, FIRE)],
                            part_hbm.at[wid, pl.ds(nfired * FIRE, FIRE)])
            for k in range(NGRP):
                sel_p[pl.ds(k * L, L)] = sel_p[pl.ds(FIRE + k * L, L)]

        nselv = jnp.where(full, nselv - FIRE, nselv)
        nsel = jnp.where(full, nsel - FIRE, nsel)
        nfired = jnp.where(full, nfired + 1, nfired)
        return nselv, nsel, nfired

    pltpu.async_copy(adj_hbm.at[:, pl.ds(0, CHUNK)], ad0_v, sem0)
    pltpu.async_copy(adj_hbm.at[:, pl.ds(CHUNK, CHUNK)], ad1_v, sem1)

    def pair_body(i, carry):
        for p in range(2):
            b = 2 * i + p
            pltpu.make_async_copy(
                adj_hbm.at[:, pl.ds(b * CHUNK, CHUNK)], ad_v[p], sems[p]
            ).wait()
            carry = lax.fori_loop(
                0, CHUNK // (NGRP * L), functools.partial(grp_body, ad_v[p]),
                carry)

            @pl.when(b + 2 < NCHUNK)
            def _refill():
                pltpu.async_copy(
                    adj_hbm.at[:, pl.ds((b + 2) * CHUNK, CHUNK)],
                    ad_v[p], sems[p])
        return carry

    assert NCHUNK % 2 == 0
    _, nsel, nfired = lax.fori_loop(
        0, NCHUNK // 2, pair_body, (jnp.zeros((L,), jnp.int32), 0, 0))

    for j in range(FIRE // G):
        @pl.when(nsel > j * G)
        def _final(j=j):
            pltpu.sync_copy(
                sel_p.at[pl.ds(j * G, G)],
                part_hbm.at[wid, pl.ds(nfired * FIRE + j * G, G)])

    count = nfired * FIRE + nsel
    cnt_v[pl.ds(0, L)] = jnp.full((L,), 1, jnp.int32) * count
    pltpu.sync_copy(cnt_v, cnt_hbm.at[wid])


_partition = pl.kernel(
    _partition_body,
    out_type=[
        jax.ShapeDtypeStruct((NW, E), jnp.int32),
        jax.ShapeDtypeStruct((NW, L), jnp.int32),
    ],
    mesh=_MESH,
    compiler_params=_SC_PARAMS,
    scratch_types=[
        pltpu.VMEM((2, CHUNK), jnp.int32),
        pltpu.VMEM((2, CHUNK), jnp.int32),
        pltpu.VMEM((CAP,), jnp.int32),
        pltpu.VMEM((L,), jnp.int32),
        pltpu.SemaphoreType.DMA,
        pltpu.SemaphoreType.DMA,
    ],
)


def _segmax_body(hp_hbm, part_hbm, cnt_hbm, out_hbm,
                 agg_v, pk0_v, pk1_v, sb0_v, sb1_v, r0_v, r1_v, cnt_v,
                 sp0, sp1, sg0, sg1, *, dact):
    wid = _wid()
    pk = (pk0_v, pk1_v)
    sb = (sb0_v, sb1_v)
    rows = (r0_v, r1_v)
    sp = (sp0, sp1)
    sg = (sg0, sg1)

    pltpu.sync_copy(cnt_hbm.at[wid], cnt_v)
    count = cnt_v[pl.ds(0, L)][0]
    nb = (count + G - 1) // G

    neg = jnp.full((L,), NEG, jnp.float32)

    def init_body(r, _):
        for k in range(dact // L):
            agg_v[r, pl.ds(k * L, L)] = neg
        return 0
    lax.fori_loop(0, NPW, init_body, 0)

    def unpack(p):
        for k in range(G // L):
            sb[p][pl.ds(k * L, L)] = pk[p][pl.ds(k * L, L)] & 0xFFFF

    def start_load(b, p):
        pltpu.async_copy(part_hbm.at[wid, pl.ds(b * G, G)],
                         pk[p].at[pl.ds(0, G)], sp[p])

    def wait_load(b, p):
        pltpu.make_async_copy(part_hbm.at[wid, pl.ds(b * G, G)],
                              pk[p].at[pl.ds(0, G)], sp[p]).wait()

    def start_gather(p):
        pltpu.async_copy(hp_hbm.at[sb[p]], rows[p], sg[p])

    def wait_gather(p):
        pltpu.make_async_copy(hp_hbm.at[sb[p]], rows[p], sg[p]).wait()

    def _upd(p, j, loc):
        nblk = dact // L
        rv = [rows[p][j, pl.ds(k * L, L)] for k in range(nblk)]
        av = [agg_v[loc, pl.ds(k * L, L)] for k in range(nblk)]
        for k in range(nblk):
            agg_v[loc, pl.ds(k * L, L)] = jnp.maximum(av[k], rv[k])

    def drain(p, nd):
        @pl.when(nd == G)
        def _full():
            def blk_body(blk, _):
                jb = blk * L
                locv = pk[p][pl.ds(jb, L)] >> 16
                for lane in range(L):
                    _upd(p, jb + lane, locv[lane])
                return 0
            lax.fori_loop(0, G // L, blk_body, 0)

        @pl.when(nd < G)
        def _partial():
            def edge_body(j, _):
                pval = pk[p][pl.ds(j, L)][0]
                _upd(p, j, pval >> 16)
                return 0
            lax.fori_loop(0, nd, edge_body, 0)

    @pl.when(nb > 0)
    def _prologue():
        pltpu.sync_copy(part_hbm.at[wid, pl.ds(0, G)], pk[0].at[pl.ds(0, G)])
        unpack(0)
        start_gather(0)

        @pl.when(nb > 1)
        def _():
            start_load(1, 1)

    def pair_body(i, _):
        for p in range(2):
            b = 2 * i + p
            q = 1 - p

            @pl.when(b < nb)
            def _do():
                @pl.when(b + 1 < nb)
                def _():
                    wait_load(b + 1, q)
                    unpack(q)
                    start_gather(q)

                wait_gather(p)
                drain(p, jnp.minimum(G, count - b * G))

                @pl.when(b + 2 < nb)
                def _():
                    start_load(b + 2, p)
        return 0

    lax.fori_loop(0, (nb + 1) // 2, pair_body, 0)
    pltpu.sync_copy(agg_v, out_hbm.at[pl.ds(wid * NPW, NPW)])


@functools.lru_cache(maxsize=None)
def _make_segmax(dact):
    return pl.kernel(
        functools.partial(_segmax_body, dact=dact),
        out_type=jax.ShapeDtypeStruct((NPAD, dact), jnp.float32),
        mesh=_MESH,
        compiler_params=_SC_PARAMS,
        scratch_types=[
            pltpu.VMEM((NPW, dact), jnp.float32),
            pltpu.VMEM((G + L,), jnp.int32),
            pltpu.VMEM((G + L,), jnp.int32),
            pltpu.VMEM((G,), jnp.int32),
            pltpu.VMEM((G,), jnp.int32),
            pltpu.VMEM((G, 128), jnp.float32),
            pltpu.VMEM((G, 128), jnp.float32),
            pltpu.VMEM((L,), jnp.int32),
            pltpu.SemaphoreType.DMA,
            pltpu.SemaphoreType.DMA,
            pltpu.SemaphoreType.DMA,
            pltpu.SemaphoreType.DMA,
        ],
    )


def _pad2(w, r, c):
    return jnp.pad(w, ((0, r - w.shape[0]), (0, c - w.shape[1])))


def kernel(x, adj, params):
    dpi_l = [128, 80, 80, 80, 80]
    dact_l = [128, 80, 80, 80, 80]
    dpo_l = [80, 80, 80, 80, 128]

    wp_l = [_pad2(params['Wp%d' % i], dpi_l[i], 128) for i in range(5)]
    bp_l = [jnp.pad(params['bp%d' % i],
                    (0, 128 - params['bp%d' % i].shape[0])).reshape(1, 128)
            for i in range(5)]

    part, cnts = _partition(adj)
    h, hp = _norm_pool(x, wp_l[0], bp_l[0])

    for i in range(5):
        dpi, dact, dpo = dpi_l[i], dact_l[i], dpo_l[i]
        ws = _pad2(params['Ws%d' % i], dpi, dpo)
        wn = _pad2(params['Wn%d' % i], dact, dpo)
        b = jnp.pad(params['b%d' % i],
                    (0, dpo - params['b%d' % i].shape[0])).reshape(1, dpo)

        agg = _make_segmax(dact)(hp, part, cnts)[:N]
        if i < 4:
            h, hp = _combine_pool(h, agg, ws, wn, b, wp_l[i + 1], bp_l[i + 1])
        else:
            h = _combine(h, agg, ws, wn, b, relu=False)

    return h
